# Initial kernel scaffold; baseline (speedup 1.0000x reference)
import jax
import jax.numpy as jnp
from jax.experimental import pallas as pl


def kernel(inputs, table):
    raise NotImplementedError("write your pallas kernel here")


# speedup vs baseline: 2.9768x; 2.9768x over previous
import functools

import jax
import jax.numpy as jnp
from jax import lax
from jax.experimental import pallas as pl
from jax.experimental.pallas import tpu as pltpu
from jax.experimental.pallas import tpu_sc as plsc

_NUM_SHARDS = 20
_ROWS_PER_SHARD = 20000
_DIM = 256
_BATCH = 16384
_HIST = 50

_LANES = 16
_NW = 32
_RW = _BATCH // _NW
_CB = 8
_LC = _CB * _HIST
_NCHUNK = _RW // _CB
_DV = _DIM // _LANES


def _flat_index(ids):
    return (ids % _NUM_SHARDS) * _ROWS_PER_SHARD + ids // _NUM_SHARDS


@functools.partial(
    pl.kernel,
    out_type=jax.ShapeDtypeStruct((_BATCH, _DIM), jnp.float32),
    mesh=plsc.VectorSubcoreMesh(core_axis_name="c", subcore_axis_name="s"),
    scratch_types=[
        pltpu.VMEM((_LC,), jnp.int32),
        pltpu.VMEM((_LC,), jnp.int32),
        pltpu.VMEM((_LC, _DIM), jnp.float32),
        pltpu.VMEM((_CB, _DIM), jnp.float32),
        pltpu.SemaphoreType.DMA,
    ],
)
def _emb_pool(ids_hbm, table_hbm, out_hbm, raw_v, fidx_v, rows_v, ostage_v, sem):
    wid = lax.axis_index("s") * 2 + lax.axis_index("c")
    base_lookup = wid * (_RW * _HIST)
    base_row = wid * _RW

    def chunk_body(_, carry):
        off, orow = carry
        off = pl.multiple_of(off, 8)
        orow = pl.multiple_of(orow, 8)
        pltpu.sync_copy(ids_hbm.at[pl.ds(off, _LC)], raw_v)
        for i in range(_LC // _LANES):
            sl = pl.ds(i * _LANES, _LANES)
            fidx_v[sl] = (raw_v[sl] % _NUM_SHARDS) * _ROWS_PER_SHARD + (raw_v[sl].astype(jnp.float32) * jnp.float32(1.0 / _NUM_SHARDS)).astype(jnp.int32)

        pltpu.async_copy(table_hbm.at[fidx_v], rows_v, sem).wait()

        for r in range(_CB):
            def red(_, st, _r=r):
                li, accs = st
                accs = tuple(
                    accs[d] + rows_v[_r * _HIST + li, pl.ds(d * _LANES, _LANES)]
                    for d in range(_DV)
                )
                return li + 1, accs
            accs = tuple(
                rows_v[r * _HIST, pl.ds(d * _LANES, _LANES)] for d in range(_DV)
            )
            _, accs = lax.fori_loop(1, _HIST, red, (jnp.int32(1), accs))
            for d in range(_DV):
                ostage_v[r, pl.ds(d * _LANES, _LANES)] = accs[d]

        pltpu.sync_copy(ostage_v, out_hbm.at[pl.ds(orow, _CB)])
        return off + _LC, orow + _CB

    lax.fori_loop(0, _NCHUNK, chunk_body, (base_lookup, base_row))


def kernel(inputs, table):
    ids = inputs.reshape(-1).astype(jnp.int32)
    tab = table.reshape(_NUM_SHARDS * _ROWS_PER_SHARD, _DIM)
    return _emb_pool(ids, tab)
